# baseline (speedup 1.0000x reference)
import functools

import jax
import jax.numpy as jnp
from jax import lax
from jax.experimental import pallas as pl
from jax.experimental.pallas import tpu as pltpu
from jax.experimental.pallas import tpu_sc as plsc


def kernel(style_ids, frame_ids, latents, latents_mu):
    S, F, D = latents.shape
    B = style_ids.shape[0]
    flat_table = latents.reshape(S * F, D)
    style_ids = style_ids.astype(jnp.int32)
    frame_ids = frame_ids.astype(jnp.int32)

    info = plsc.get_sparse_core_info()
    NC, NS, L = info.num_cores, info.num_subcores, info.num_lanes
    NW = NC * NS
    b_per_w = B // NW
    H = b_per_w // 2

    mesh = plsc.VectorSubcoreMesh(core_axis_name="c", subcore_axis_name="s")

    @functools.partial(
        pl.kernel,
        mesh=mesh,
        out_type=jax.ShapeDtypeStruct((B, D), jnp.float32),
        scratch_types=[
            pltpu.VMEM((b_per_w,), jnp.int32),
            pltpu.VMEM((b_per_w,), jnp.int32),
            pltpu.VMEM((b_per_w // 4, D), jnp.float32),
            pltpu.VMEM((b_per_w // 4, D), jnp.float32),
            pltpu.VMEM((b_per_w // 4, D), jnp.float32),
            pltpu.VMEM((b_per_w // 4, D), jnp.float32),
            pltpu.SemaphoreType.DMA,
            pltpu.SemaphoreType.DMA,
            pltpu.SemaphoreType.DMA,
            pltpu.SemaphoreType.DMA,
        ],
    )
    def run(style_hbm, frame_hbm, table_hbm, mu_hbm, out_hbm,
            sty_v, idx_v, r0, r1, r2, r3, s0, s1, s2, s3):
        wid = lax.axis_index("s") * NC + lax.axis_index("c")
        base = wid * b_per_w
        Q = b_per_w // 4
        rows = (r0, r1, r2, r3)
        sems = (s0, s1, s2, s3)
        c_sty = pltpu.async_copy(style_hbm.at[pl.ds(base, b_per_w)], sty_v, s0)
        c_frm = pltpu.async_copy(frame_hbm.at[pl.ds(base, b_per_w)], idx_v, s1)
        c_sty.wait()
        c_frm.wait()

        def flat_body(i, _):
            sl = pl.ds(i * L, L)
            idx_v[sl] = sty_v[sl] * F + idx_v[sl]
            return 0
        lax.fori_loop(0, b_per_w // L, flat_body, 0)

        gs = [pltpu.async_copy(table_hbm.at[idx_v.at[pl.ds(q * Q, Q)]],
                               rows[q], sems[q]) for q in range(4)]
        adds = []
        for q in range(4):
            gs[q].wait()
            adds.append(pltpu.async_copy(mu_hbm.at[sty_v.at[pl.ds(q * Q, Q)]],
                                         rows[q], sems[q], add=True))
        ws = []
        for q in range(4):
            adds[q].wait()
            ws.append(pltpu.async_copy(rows[q],
                                       out_hbm.at[pl.ds(base + q * Q, Q)],
                                       sems[q]))
        for q in range(4):
            ws[q].wait()

    return run(style_ids, frame_ids, flat_table, latents_mu)


# speedup vs baseline: 1.1319x; 1.1319x over previous
import functools

import jax
import jax.numpy as jnp
from jax import lax
from jax.experimental import pallas as pl
from jax.experimental.pallas import tpu as pltpu
from jax.experimental.pallas import tpu_sc as plsc


def kernel(style_ids, frame_ids, latents, latents_mu):
    S, F, D = latents.shape
    B = style_ids.shape[0]
    flat_table = latents.reshape(S * F, D)
    style_ids = style_ids.astype(jnp.int32)
    frame_ids = frame_ids.astype(jnp.int32)

    info = plsc.get_sparse_core_info()
    NC, NS, L = info.num_cores, info.num_subcores, info.num_lanes
    NW = NC * NS
    b_per_w = B // NW
    H = b_per_w // 2

    mesh = plsc.VectorSubcoreMesh(core_axis_name="c", subcore_axis_name="s")

    @functools.partial(
        pl.kernel,
        mesh=mesh,
        out_type=jax.ShapeDtypeStruct((B, D), jnp.float32),
        scratch_types=[
            pltpu.VMEM((b_per_w,), jnp.int32),
            pltpu.VMEM((b_per_w,), jnp.int32),
            pltpu.VMEM((H, D), jnp.float32),
            pltpu.VMEM((H, D), jnp.float32),
            pltpu.VMEM_SHARED((S, D), jnp.float32),
            pltpu.SemaphoreType.DMA,
            pltpu.SemaphoreType.DMA,
            pltpu.SemaphoreType.DMA,
        ],
    )
    def run(style_hbm, frame_hbm, table_hbm, mu_hbm, out_hbm,
            sty_v, idx_v, buf0, buf1, mu_sh, sem_a, sem_b, sem_w):
        sid = lax.axis_index("s")
        wid = sid * NC + lax.axis_index("c")
        base = wid * b_per_w
        c_sty = pltpu.async_copy(style_hbm.at[pl.ds(base, b_per_w)], sty_v,
                                 sem_a)
        c_frm = pltpu.async_copy(frame_hbm.at[pl.ds(base, b_per_w)], idx_v,
                                 sem_b)

        n_stage = S // 5
        @pl.when(sid < 5)
        def _stage():
            pltpu.sync_copy(mu_hbm.at[pl.ds(sid * n_stage, n_stage)],
                            mu_sh.at[pl.ds(sid * n_stage, n_stage)])

        c_sty.wait()
        c_frm.wait()

        def flat_body(i, _):
            sl = pl.ds(i * L, L)
            idx_v[sl] = sty_v[sl] * F + idx_v[sl]
            return 0
        lax.fori_loop(0, b_per_w // L, flat_body, 0)

        g0 = pltpu.async_copy(table_hbm.at[idx_v.at[pl.ds(0, H)]], buf0, sem_a)
        g1 = pltpu.async_copy(table_hbm.at[idx_v.at[pl.ds(H, H)]], buf1, sem_b)
        plsc.subcore_barrier()
        g0.wait()
        a0 = pltpu.async_copy(mu_sh.at[sty_v.at[pl.ds(0, H)]], buf0, sem_a,
                              add=True)
        a0.wait()
        w0 = pltpu.async_copy(buf0, out_hbm.at[pl.ds(base, H)], sem_w)
        g1.wait()
        a1 = pltpu.async_copy(mu_sh.at[sty_v.at[pl.ds(H, H)]], buf1, sem_b,
                              add=True)
        a1.wait()
        w0.wait()
        pltpu.sync_copy(buf1, out_hbm.at[pl.ds(base + H, H)])

    return run(style_ids, frame_ids, flat_table, latents_mu)


# speedup vs baseline: 1.2061x; 1.0655x over previous
import functools

import jax
import jax.numpy as jnp
from jax import lax
from jax.experimental import pallas as pl
from jax.experimental.pallas import tpu as pltpu
from jax.experimental.pallas import tpu_sc as plsc


def kernel(style_ids, frame_ids, latents, latents_mu):
    S, F, D = latents.shape
    B = style_ids.shape[0]
    flat_table = latents.reshape(S * F, D)
    style_ids = style_ids.astype(jnp.int32)
    frame_ids = frame_ids.astype(jnp.int32)

    info = plsc.get_sparse_core_info()
    NC, NS, L = info.num_cores, info.num_subcores, info.num_lanes
    NW = NC * NS
    b_per_w = B // NW
    H = b_per_w // 2

    mesh = plsc.VectorSubcoreMesh(core_axis_name="c", subcore_axis_name="s")

    @functools.partial(
        pl.kernel,
        mesh=mesh,
        out_type=jax.ShapeDtypeStruct((B, D), jnp.float32),
        scratch_types=[
            pltpu.VMEM((b_per_w,), jnp.int32),
            pltpu.VMEM((b_per_w,), jnp.int32),
            pltpu.VMEM((b_per_w // 4, D), jnp.float32),
            pltpu.VMEM((b_per_w // 4, D), jnp.float32),
            pltpu.VMEM((b_per_w // 4, D), jnp.float32),
            pltpu.VMEM((b_per_w // 4, D), jnp.float32),
            pltpu.VMEM_SHARED((S, D), jnp.float32),
            pltpu.SemaphoreType.DMA,
            pltpu.SemaphoreType.DMA,
            pltpu.SemaphoreType.DMA,
            pltpu.SemaphoreType.DMA,
        ],
    )
    def run(style_hbm, frame_hbm, table_hbm, mu_hbm, out_hbm,
            sty_v, idx_v, b0, b1, b2, b3, mu_sh, s0, s1, s2, s3):
        sid = lax.axis_index("s")
        wid = sid * NC + lax.axis_index("c")
        base = wid * b_per_w
        Q = b_per_w // 4
        bufs = (b0, b1, b2, b3)
        sems = (s0, s1, s2, s3)
        c_sty = pltpu.async_copy(style_hbm.at[pl.ds(base, b_per_w)], sty_v, s0)
        c_frm = pltpu.async_copy(frame_hbm.at[pl.ds(base, b_per_w)], idx_v, s1)

        n_stage = S // 5
        @pl.when(sid < 5)
        def _stage():
            pltpu.sync_copy(mu_hbm.at[pl.ds(sid * n_stage, n_stage)],
                            mu_sh.at[pl.ds(sid * n_stage, n_stage)])

        c_sty.wait()
        c_frm.wait()

        def flat_body(i, _):
            sl = pl.ds(i * L, L)
            idx_v[sl] = sty_v[sl] * F + idx_v[sl]
            return 0
        lax.fori_loop(0, b_per_w // L, flat_body, 0)

        def gather(q):
            return pltpu.async_copy(
                table_hbm.at[idx_v.at[pl.ds(q * Q, Q)]], bufs[q], sems[q])

        def mu_add(q):
            return pltpu.async_copy(
                mu_sh.at[sty_v.at[pl.ds(q * Q, Q)]], bufs[q], sems[q],
                add=True)

        def write(q):
            return pltpu.async_copy(
                bufs[q], out_hbm.at[pl.ds(base + q * Q, Q)], sems[q])

        gs = {0: gather(0), 1: gather(1)}
        plsc.subcore_barrier()
        adds = {}
        ws = {}
        for q in range(4):
            gs[q].wait()
            adds[q] = mu_add(q)
            if q + 2 < 4:
                gs[q + 2] = gather(q + 2)
            adds[q].wait()
            ws[q] = write(q)
        for q in range(4):
            ws[q].wait()

    return run(style_ids, frame_ids, flat_table, latents_mu)
